# baseline (speedup 1.0000x reference)
import dataclasses
import functools

import jax
import jax.numpy as jnp
from jax import lax
from jax.experimental import pallas as pl
from jax.experimental.pallas import tpu as pltpu
from jax.experimental.pallas import tpu_sc as plsc

N_NODES = 10000
N_EDGES = 320000
D = 128
NC = 2
NS = 16
NW = NC * NS
EPW = N_EDGES // NW
C = 40
NCH = EPW // C
SS = 50
NSS = NCH // SS
NBUF = 5
ZR = 624
TAIL = N_NODES - NS * ZR
LANES = 16

_mesh = plsc.VectorSubcoreMesh(core_axis_name="c", subcore_axis_name="s")

_cp = pltpu.CompilerParams()
if "needs_layout_passes" in pltpu.CompilerParams.__dataclass_fields__:
    _cp = dataclasses.replace(_cp, needs_layout_passes=False)


@functools.partial(
    pl.kernel,
    out_type=jax.ShapeDtypeStruct((NC, N_NODES, D), jnp.float32),
    mesh=_mesh,
    compiler_params=_cp,
    scratch_types=[
        pltpu.VMEM((SS, C), jnp.int32),
        pltpu.VMEM((SS, C), jnp.int32),
        pltpu.VMEM((SS, C), jnp.float32),
        *[pltpu.VMEM((C, D), jnp.float32) for _ in range(NBUF)],
        pltpu.VMEM_SHARED((N_NODES, D), jnp.float32),
        *[pltpu.SemaphoreType.DMA for _ in range(3 * NBUF)],
    ],
)
def _sc_aggregate(x_hbm, src_hbm, dst_hbm, val_hbm, out_hbm,
                  src_v, dst_v, val_v, *rest):
    rows = rest[:NBUF]
    acc = rest[NBUF]
    gsem = rest[NBUF + 1:2 * NBUF + 1]
    gsem2 = rest[2 * NBUF + 1:3 * NBUF + 1]
    ssem = rest[3 * NBUF + 1:]
    rows0_v = rows[0]
    CA = 24
    c = lax.axis_index("c")
    s = lax.axis_index("s")
    wid = c * NS + s

    zero16 = jnp.zeros((LANES,), jnp.float32)

    @pl.loop(0, C)
    def _(r):
        for q in range(D // LANES):
            rows0_v[r, pl.ds(q * LANES, LANES)] = zero16

    base = s * ZR

    @pl.loop(0, (ZR // C) * C, step=C)
    def _(r0):
        pltpu.sync_copy(rows0_v, acc.at[pl.ds(base + r0, C)])

    ztail = ZR % C
    if ztail:
        pltpu.sync_copy(rows0_v.at[pl.ds(0, ztail)],
                        acc.at[pl.ds(base + (ZR // C) * C, ztail)])

    @pl.when(s == NS - 1)
    def _():
        pltpu.sync_copy(rows0_v.at[pl.ds(0, TAIL)],
                        acc.at[pl.ds(NS * ZR, TAIL)])

    plsc.subcore_barrier()

    def scale(rows_ref, k):
        @plsc.parallel_loop(0, C, step=1, unroll=2)
        def _(r):
            kk = jnp.full((LANES,), k, jnp.int32)
            rr = jnp.full((LANES,), r, jnp.int32)
            v16 = plsc.load_gather(val_v, [kk, rr])
            for q in range(D // LANES):
                sl = pl.ds(q * LANES, LANES)
                rows_ref[r, sl] = rows_ref[r, sl] * v16

    @pl.loop(0, NSS)
    def _(g):
        pltpu.sync_copy(src_hbm.at[wid].at[g], src_v)
        pltpu.sync_copy(dst_hbm.at[wid].at[g], dst_v)
        pltpu.sync_copy(val_hbm.at[wid].at[g], val_v)

        def start_gather(k, nb):
            ha = pltpu.async_copy(
                x_hbm.at[src_v.at[k].at[pl.ds(0, CA)]],
                rows[nb].at[pl.ds(0, CA)], gsem[nb])
            hb = pltpu.async_copy(
                x_hbm.at[src_v.at[k].at[pl.ds(CA, C - CA)]],
                rows[nb].at[pl.ds(CA, C - CA)], gsem2[nb])
            return ha, hb

        gh = [None] * SS
        sh = [None] * SS
        for k in range(NBUF - 1):
            gh[k] = start_gather(k, k)
        for k in range(SS):
            b = k % NBUF
            gh[k][0].wait()
            gh[k][1].wait()
            scale(rows[b], k)
            sh[k] = pltpu.async_copy(rows[b], acc.at[dst_v.at[k]],
                                     ssem[b], add=True)
            if k + NBUF - 1 < SS:
                nb = (k + NBUF - 1) % NBUF
                if k >= 1:
                    sh[k - 1].wait()
                gh[k + NBUF - 1] = start_gather(k + NBUF - 1, nb)
        for k in range(max(0, SS - NBUF), SS):
            sh[k].wait()

    plsc.subcore_barrier()
    pltpu.sync_copy(acc.at[pl.ds(base, ZR)],
                    out_hbm.at[c].at[pl.ds(base, ZR)])

    @pl.when(s == NS - 1)
    def _():
        pltpu.sync_copy(acc.at[pl.ds(NS * ZR, TAIL)],
                        out_hbm.at[c].at[pl.ds(NS * ZR, TAIL)])


_BLK = 1000


def _mm_body(p_ref, w_ref, b_ref, o_ref):
    agg = p_ref[0] + p_ref[1]
    o_ref[...] = jnp.dot(agg, w_ref[...],
                         preferred_element_type=jnp.float32,
                         precision=lax.Precision.HIGHEST) + b_ref[...]


def _tc_matmul(partials, weight, bias2d):
    return pl.pallas_call(
        _mm_body,
        grid=(N_NODES // _BLK,),
        in_specs=[
            pl.BlockSpec((NC, _BLK, D), lambda i: (0, i, 0)),
            pl.BlockSpec((D, D), lambda i: (0, 0)),
            pl.BlockSpec((1, D), lambda i: (0, 0)),
        ],
        out_specs=pl.BlockSpec((_BLK, D), lambda i: (i, 0)),
        out_shape=jax.ShapeDtypeStruct((N_NODES, D), jnp.float32),
    )(partials, weight, bias2d)


def kernel(x, edge_index, adj_values, weight, bias):
    ei = edge_index.astype(jnp.int32)
    src = ei[1].reshape(NW, NSS, SS, C)
    dst = ei[0].reshape(NW, NSS, SS, C)
    vals = adj_values.reshape(NW, NSS, SS, C)
    partials = _sc_aggregate(x, src, dst, vals)
    return _tc_matmul(partials, weight, bias.reshape(1, D))


# speedup vs baseline: 1.0008x; 1.0008x over previous
import dataclasses
import functools

import jax
import jax.numpy as jnp
from jax import lax
from jax.experimental import pallas as pl
from jax.experimental.pallas import tpu as pltpu
from jax.experimental.pallas import tpu_sc as plsc

N_NODES = 10000
N_EDGES = 320000
D = 128
NC = 2
NS = 16
NW = NC * NS
EPW = N_EDGES // NW
C = 40
NCH = EPW // C
SS = 50
NSS = NCH // SS
NBUF = 5
ZR = 624
TAIL = N_NODES - NS * ZR
LANES = 16

_mesh = plsc.VectorSubcoreMesh(core_axis_name="c", subcore_axis_name="s")

_cp = pltpu.CompilerParams()
if "needs_layout_passes" in pltpu.CompilerParams.__dataclass_fields__:
    _cp = dataclasses.replace(_cp, needs_layout_passes=False)


@functools.partial(
    pl.kernel,
    out_type=jax.ShapeDtypeStruct((NC, N_NODES, D), jnp.float32),
    mesh=_mesh,
    compiler_params=_cp,
    scratch_types=[
        pltpu.VMEM((SS, C), jnp.int32),
        pltpu.VMEM((SS, C), jnp.int32),
        pltpu.VMEM((SS, C), jnp.float32),
        *[pltpu.VMEM((C, D), jnp.float32) for _ in range(NBUF)],
        pltpu.VMEM_SHARED((N_NODES, D), jnp.float32),
        *[pltpu.SemaphoreType.DMA for _ in range(2 * NBUF)],
    ],
)
def _sc_aggregate(x_hbm, src_hbm, dst_hbm, val_hbm, out_hbm,
                  src_v, dst_v, val_v, *rest):
    rows = rest[:NBUF]
    acc = rest[NBUF]
    gsem = rest[NBUF + 1:2 * NBUF + 1]
    ssem = rest[2 * NBUF + 1:]
    rows0_v = rows[0]
    c = lax.axis_index("c")
    s = lax.axis_index("s")
    wid = c * NS + s

    zero16 = jnp.zeros((LANES,), jnp.float32)

    @pl.loop(0, C)
    def _(r):
        for q in range(D // LANES):
            rows0_v[r, pl.ds(q * LANES, LANES)] = zero16

    base = s * ZR

    @pl.loop(0, (ZR // C) * C, step=C)
    def _(r0):
        pltpu.sync_copy(rows0_v, acc.at[pl.ds(base + r0, C)])

    ztail = ZR % C
    if ztail:
        pltpu.sync_copy(rows0_v.at[pl.ds(0, ztail)],
                        acc.at[pl.ds(base + (ZR // C) * C, ztail)])

    @pl.when(s == NS - 1)
    def _():
        pltpu.sync_copy(rows0_v.at[pl.ds(0, TAIL)],
                        acc.at[pl.ds(NS * ZR, TAIL)])

    plsc.subcore_barrier()

    def scale(rows_ref, k):
        @plsc.parallel_loop(0, C, step=1, unroll=2)
        def _(r):
            kk = jnp.full((LANES,), k, jnp.int32)
            rr = jnp.full((LANES,), r, jnp.int32)
            v16 = plsc.load_gather(val_v, [kk, rr])
            for q in range(D // LANES):
                sl = pl.ds(q * LANES, LANES)
                rows_ref[r, sl] = rows_ref[r, sl] * v16

    @pl.loop(0, NSS)
    def _(g):
        pltpu.sync_copy(src_hbm.at[wid].at[g], src_v)
        pltpu.sync_copy(dst_hbm.at[wid].at[g], dst_v)
        pltpu.sync_copy(val_hbm.at[wid].at[g], val_v)

        gh = [None] * SS
        sh = [None] * SS
        for k in range(NBUF - 1):
            gh[k] = pltpu.async_copy(x_hbm.at[src_v.at[k]], rows[k],
                                     gsem[k])
        for k in range(SS):
            b = k % NBUF
            gh[k].wait()
            scale(rows[b], k)
            sh[k] = pltpu.async_copy(rows[b], acc.at[dst_v.at[k]],
                                     ssem[b], add=True)
            if k + NBUF - 1 < SS:
                nb = (k + NBUF - 1) % NBUF
                if k >= 1:
                    sh[k - 1].wait()
                gh[k + NBUF - 1] = pltpu.async_copy(
                    x_hbm.at[src_v.at[k + NBUF - 1]], rows[nb], gsem[nb])
        for k in range(max(0, SS - NBUF), SS):
            sh[k].wait()

    plsc.subcore_barrier()
    pltpu.sync_copy(acc.at[pl.ds(base, ZR)],
                    out_hbm.at[c].at[pl.ds(base, ZR)])

    @pl.when(s == NS - 1)
    def _():
        pltpu.sync_copy(acc.at[pl.ds(NS * ZR, TAIL)],
                        out_hbm.at[c].at[pl.ds(NS * ZR, TAIL)])


_BLK = 1000


def _mm_body(p_ref, w_ref, b_ref, o_ref):
    agg = p_ref[0] + p_ref[1]
    o_ref[...] = jnp.dot(agg, w_ref[...],
                         preferred_element_type=jnp.float32,
                         precision=lax.Precision.HIGHEST) + b_ref[...]


def _tc_matmul(partials, weight, bias2d):
    return pl.pallas_call(
        _mm_body,
        grid=(N_NODES // _BLK,),
        in_specs=[
            pl.BlockSpec((NC, _BLK, D), lambda i: (0, i, 0)),
            pl.BlockSpec((D, D), lambda i: (0, 0)),
            pl.BlockSpec((1, D), lambda i: (0, 0)),
        ],
        out_specs=pl.BlockSpec((_BLK, D), lambda i: (i, 0)),
        out_shape=jax.ShapeDtypeStruct((N_NODES, D), jnp.float32),
    )(partials, weight, bias2d)


def kernel(x, edge_index, adj_values, weight, bias):
    ei = edge_index.astype(jnp.int32)
    src = ei[1].reshape(NW, NSS, SS, C)
    dst = ei[0].reshape(NW, NSS, SS, C)
    vals = adj_values.reshape(NW, NSS, SS, C)
    partials = _sc_aggregate(x, src, dst, vals)
    return _tc_matmul(partials, weight, bias.reshape(1, D))


# speedup vs baseline: 1.0481x; 1.0472x over previous
import dataclasses
import functools

import jax
import jax.numpy as jnp
from jax import lax
from jax.experimental import pallas as pl
from jax.experimental.pallas import tpu as pltpu
from jax.experimental.pallas import tpu_sc as plsc

N_NODES = 10000
N_EDGES = 320000
D = 128
NC = 2
NS = 16
NW = NC * NS
EPW = N_EDGES // NW
C = 40
NCH = EPW // C
SS = 50
NSS = NCH // SS
NBUF = 5
ZR = 624
TAIL = N_NODES - NS * ZR
LANES = 16

_mesh = plsc.VectorSubcoreMesh(core_axis_name="c", subcore_axis_name="s")

_cp = pltpu.CompilerParams()
if "needs_layout_passes" in pltpu.CompilerParams.__dataclass_fields__:
    _cp = dataclasses.replace(_cp, needs_layout_passes=False)


@functools.partial(
    pl.kernel,
    out_type=jax.ShapeDtypeStruct((NC, N_NODES, D), jnp.float32),
    mesh=_mesh,
    compiler_params=_cp,
    scratch_types=[
        pltpu.VMEM((SS, C), jnp.int32),
        pltpu.VMEM((SS, C), jnp.int32),
        pltpu.VMEM((SS, C), jnp.float32),
        *[pltpu.VMEM((C, D), jnp.float32) for _ in range(NBUF)],
        pltpu.VMEM_SHARED((N_NODES, D), jnp.float32),
        *[pltpu.SemaphoreType.DMA for _ in range(2 * NBUF)],
    ],
)
def _sc_aggregate(x_hbm, src_hbm, dst_hbm, val_hbm, out_hbm,
                  src_v, dst_v, val_v, *rest):
    rows = rest[:NBUF]
    acc = rest[NBUF]
    gsem = rest[NBUF + 1:2 * NBUF + 1]
    ssem = rest[2 * NBUF + 1:]
    rows0_v = rows[0]
    c = lax.axis_index("c")
    s = lax.axis_index("s")
    wid = c * NS + s

    zero16 = jnp.zeros((LANES,), jnp.float32)

    @pl.loop(0, C)
    def _(r):
        for q in range(D // LANES):
            rows0_v[r, pl.ds(q * LANES, LANES)] = zero16

    base = s * ZR

    @pl.loop(0, (ZR // C) * C, step=C)
    def _(r0):
        pltpu.sync_copy(rows0_v, acc.at[pl.ds(base + r0, C)])

    ztail = ZR % C
    if ztail:
        pltpu.sync_copy(rows0_v.at[pl.ds(0, ztail)],
                        acc.at[pl.ds(base + (ZR // C) * C, ztail)])

    @pl.when(s == NS - 1)
    def _():
        pltpu.sync_copy(rows0_v.at[pl.ds(0, TAIL)],
                        acc.at[pl.ds(NS * ZR, TAIL)])

    plsc.subcore_barrier()

    def scale(rows_ref, k):
        @plsc.parallel_loop(0, C, step=1, unroll=2)
        def _(r):
            kk = jnp.full((LANES,), k, jnp.int32)
            rr = jnp.full((LANES,), r, jnp.int32)
            v16 = plsc.load_gather(val_v, [kk, rr])
            for q in range(D // LANES):
                sl = pl.ds(q * LANES, LANES)
                rows_ref[r, sl] = rows_ref[r, sl] * v16

    @pl.loop(0, NSS)
    def _(g):
        st0 = pltpu.async_copy(src_hbm.at[wid].at[g], src_v, gsem[0])
        st1 = pltpu.async_copy(dst_hbm.at[wid].at[g], dst_v, gsem[1])
        st2 = pltpu.async_copy(val_hbm.at[wid].at[g], val_v, gsem[2])
        st0.wait()
        st1.wait()
        st2.wait()

        gh = [None] * SS
        sh = [None] * SS
        for k in range(NBUF - 1):
            gh[k] = pltpu.async_copy(x_hbm.at[src_v.at[k]], rows[k],
                                     gsem[k])
        for k in range(SS):
            b = k % NBUF
            gh[k].wait()
            scale(rows[b], k)
            sh[k] = pltpu.async_copy(rows[b], acc.at[dst_v.at[k]],
                                     ssem[b], add=True)
            if k + NBUF - 1 < SS:
                nb = (k + NBUF - 1) % NBUF
                if k >= 1:
                    sh[k - 1].wait()
                gh[k + NBUF - 1] = pltpu.async_copy(
                    x_hbm.at[src_v.at[k + NBUF - 1]], rows[nb], gsem[nb])
        for k in range(max(0, SS - NBUF), SS):
            sh[k].wait()

    plsc.subcore_barrier()
    pltpu.sync_copy(acc.at[pl.ds(base, ZR)],
                    out_hbm.at[c].at[pl.ds(base, ZR)])

    @pl.when(s == NS - 1)
    def _():
        pltpu.sync_copy(acc.at[pl.ds(NS * ZR, TAIL)],
                        out_hbm.at[c].at[pl.ds(NS * ZR, TAIL)])


_BLK = 1000


def _mm_body(p_ref, w_ref, b_ref, o_ref):
    agg = p_ref[0] + p_ref[1]
    o_ref[...] = jnp.dot(agg, w_ref[...],
                         preferred_element_type=jnp.float32) + b_ref[...]


def _tc_matmul(partials, weight, bias2d):
    return pl.pallas_call(
        _mm_body,
        grid=(N_NODES // _BLK,),
        in_specs=[
            pl.BlockSpec((NC, _BLK, D), lambda i: (0, i, 0)),
            pl.BlockSpec((D, D), lambda i: (0, 0)),
            pl.BlockSpec((1, D), lambda i: (0, 0)),
        ],
        out_specs=pl.BlockSpec((_BLK, D), lambda i: (i, 0)),
        out_shape=jax.ShapeDtypeStruct((N_NODES, D), jnp.float32),
    )(partials, weight, bias2d)


def kernel(x, edge_index, adj_values, weight, bias):
    ei = edge_index.astype(jnp.int32)
    src = ei[1].reshape(NW, NSS, SS, C)
    dst = ei[0].reshape(NW, NSS, SS, C)
    vals = adj_values.reshape(NW, NSS, SS, C)
    partials = _sc_aggregate(x, src, dst, vals)
    return _tc_matmul(partials, weight, bias.reshape(1, D))
